# baseline (speedup 1.0000x reference)
import functools
import math

import jax
import jax.numpy as jnp
from jax import lax
from jax.experimental import pallas as pl
from jax.experimental.pallas import tpu as pltpu
from jax.experimental.pallas import tpu_sc as plsc

D_MODEL = 512
MAX_SEQ = 200
VOCAB = 76
VOCAB_PAD = 80
BATCH = 1024

_info = plsc.get_sparse_core_info()
_NUM_CORES = _info.num_cores
_NUM_SUBCORES = _info.num_subcores
_NUM_WORKERS = _NUM_CORES * _NUM_SUBCORES
_LANES = _info.num_lanes

N_ROWS = BATCH * MAX_SEQ
B_SC = 512
B_TC = BATCH - B_SC
ROWS_PER_W = B_SC * MAX_SEQ // _NUM_WORKERS
CHUNK = 40
N_CHUNKS = ROWS_PER_W // CHUNK
NBUF = 5


def _positional_encoding():
    even_i = jnp.arange(0, D_MODEL, 2, dtype=jnp.float32)
    denominator = jnp.power(10000.0, even_i / D_MODEL)
    position = jnp.arange(0, MAX_SEQ, 1, dtype=jnp.float32).reshape(MAX_SEQ, 1)
    even_pe = jnp.sin(position / denominator)
    odd_pe = jnp.cos(position / denominator)
    return jnp.stack([even_pe, odd_pe], axis=2).reshape(MAX_SEQ, D_MODEL)


def _fuse_body(table_ref, pe_ref, out_ref):
    out_ref[...] = (
        table_ref[...] * math.sqrt(float(D_MODEL)) + pe_ref[...][:, None, :]
    )


_build_fused = pl.pallas_call(
    _fuse_body,
    out_shape=jax.ShapeDtypeStruct((MAX_SEQ, VOCAB_PAD, D_MODEL), jnp.float32),
)

SB = 8


def _tc_body(tok_ref, comb_ref, posoh_ref, out_ref):
    t = tok_ref[...]
    oh = (t == lax.broadcasted_iota(jnp.int32, (1, VOCAB_PAD), 1)).astype(
        jnp.bfloat16
    )
    ohc = jnp.concatenate([oh, posoh_ref[...]], axis=1)
    acc = jnp.dot(ohc, comb_ref[...], preferred_element_type=jnp.float32)
    out_ref[...] = jnp.reshape(acc, (SB, MAX_SEQ, D_MODEL))


_tc_fill = pl.pallas_call(
    _tc_body,
    grid=(B_TC // SB,),
    in_specs=[
        pl.BlockSpec((SB * MAX_SEQ, 1), lambda b: (b, 0)),
        pl.BlockSpec((VOCAB_PAD + MAX_SEQ, D_MODEL), lambda b: (0, 0)),
        pl.BlockSpec((SB * MAX_SEQ, MAX_SEQ), lambda b: (0, 0)),
    ],
    out_specs=pl.BlockSpec((SB, MAX_SEQ, D_MODEL), lambda b: (b, 0, 0)),
    out_shape=jax.ShapeDtypeStruct((B_TC, MAX_SEQ, D_MODEL), jnp.float32),
)

_mesh = plsc.VectorSubcoreMesh(core_axis_name="c", subcore_axis_name="s")


@functools.partial(
    pl.kernel,
    out_type=jax.ShapeDtypeStruct((B_SC * MAX_SEQ, D_MODEL), jnp.float32),
    mesh=_mesh,
    scratch_types=[
        pltpu.VMEM((ROWS_PER_W,), jnp.int32),
        [pltpu.VMEM((CHUNK, D_MODEL), jnp.float32) for _ in range(NBUF)],
        [pltpu.SemaphoreType.DMA for _ in range(NBUF)],
        [pltpu.SemaphoreType.DMA for _ in range(NBUF)],
    ],
)
def _gather_kernel(tok_hbm, fused_hbm, out_hbm, idx_v, bufs, gsems, ssems):
    wid = lax.axis_index("s") * _NUM_CORES + lax.axis_index("c")
    base = wid * ROWS_PER_W
    pltpu.sync_copy(tok_hbm.at[pl.ds(base, ROWS_PER_W)], idx_v)

    lanes = lax.iota(jnp.int32, _LANES)

    def idx_body(j, carry):
        o = j * _LANES
        tok = idx_v[pl.ds(o, _LANES)]
        pos = jnp.remainder(o + lanes, MAX_SEQ)
        idx_v[pl.ds(o, _LANES)] = pos * VOCAB_PAD + tok
        return carry

    lax.fori_loop(0, ROWS_PER_W // _LANES, idx_body, 0)

    def fire_gather(c, b):
        pltpu.async_copy(
            fused_hbm.at[idx_v.at[pl.ds(c * CHUNK, CHUNK)]], bufs[b], gsems[b]
        )

    def wait_gather(b):
        pltpu.make_async_copy(
            out_hbm.at[pl.ds(base, CHUNK)], bufs[b], gsems[b]
        ).wait()

    def fire_scatter(c, b):
        pltpu.async_copy(
            bufs[b], out_hbm.at[pl.ds(base + c * CHUNK, CHUNK)], ssems[b]
        )

    def wait_scatter(b):
        pltpu.make_async_copy(
            bufs[b], out_hbm.at[pl.ds(base, CHUNK)], ssems[b]
        ).wait()

    fire_gather(0, 0)
    fire_gather(1, 1)
    for c in (0, 1, 2):
        wait_gather(c)
        fire_scatter(c, c)
        fire_gather(c + 2, c + 2)

    def chunk_body(g, carry):
        for k in range(NBUF):
            c = 3 + g * NBUF + k
            b = (3 + k) % NBUF
            b2 = (5 + k) % NBUF
            wait_gather(b)
            fire_scatter(c, b)
            wait_scatter(b2)
            fire_gather(c + 2, b2)
        return carry

    lax.fori_loop(0, (N_CHUNKS - 5) // NBUF, chunk_body, 0)

    for c in (N_CHUNKS - 2, N_CHUNKS - 1):
        b = c % NBUF
        wait_gather(b)
        fire_scatter(c, b)
    for b in range(NBUF):
        wait_scatter(b)


def kernel(token_ids, embedding_table):
    tok_flat = token_ids.reshape(-1).astype(jnp.int32)
    table_pad = jnp.pad(embedding_table, ((0, VOCAB_PAD - VOCAB), (0, 0)))
    pe = _positional_encoding()
    fused = _build_fused(table_pad, pe).reshape(MAX_SEQ * VOCAB_PAD, D_MODEL)
    sc_out = _gather_kernel(tok_flat, fused).reshape(B_SC, MAX_SEQ, D_MODEL)
    scaled = table_pad * math.sqrt(float(D_MODEL))
    comb = jnp.concatenate(
        [scaled.astype(jnp.bfloat16), pe.astype(jnp.bfloat16)], axis=0
    )
    rows = jnp.arange(SB * MAX_SEQ, dtype=jnp.int32)
    posoh = (
        (rows % MAX_SEQ)[:, None] == jnp.arange(MAX_SEQ, dtype=jnp.int32)[None]
    ).astype(jnp.bfloat16)
    tok_tc = token_ids[B_SC:].reshape(B_TC * MAX_SEQ, 1).astype(jnp.int32)
    tc_out = _tc_fill(tok_tc, comb, posoh)
    return jnp.concatenate([sc_out, tc_out], axis=0)


# speedup vs baseline: 1.8805x; 1.8805x over previous
import functools
import math

import jax
import jax.numpy as jnp
from jax import lax
from jax.experimental import pallas as pl
from jax.experimental.pallas import tpu as pltpu
from jax.experimental.pallas import tpu_sc as plsc

D_MODEL = 512
MAX_SEQ = 200
VOCAB = 76
VOCAB_PAD = 80
BATCH = 1024

_info = plsc.get_sparse_core_info()
_NUM_CORES = _info.num_cores
_NUM_SUBCORES = _info.num_subcores
_NUM_WORKERS = _NUM_CORES * _NUM_SUBCORES
_LANES = _info.num_lanes

N_ROWS = BATCH * MAX_SEQ
B_SC = 512
B_TC = BATCH - B_SC
ROWS_PER_W = B_SC * MAX_SEQ // _NUM_WORKERS
CHUNK = 40
N_CHUNKS = ROWS_PER_W // CHUNK
NBUF = 5


def _positional_encoding():
    even_i = jnp.arange(0, D_MODEL, 2, dtype=jnp.float32)
    denominator = jnp.power(10000.0, even_i / D_MODEL)
    position = jnp.arange(0, MAX_SEQ, 1, dtype=jnp.float32).reshape(MAX_SEQ, 1)
    even_pe = jnp.sin(position / denominator)
    odd_pe = jnp.cos(position / denominator)
    return jnp.stack([even_pe, odd_pe], axis=2).reshape(MAX_SEQ, D_MODEL)


def _fuse_body(table_ref, pe_ref, out_ref):
    out_ref[...] = (
        table_ref[...] * math.sqrt(float(D_MODEL)) + pe_ref[...][:, None, :]
    )


_build_fused = pl.pallas_call(
    _fuse_body,
    out_shape=jax.ShapeDtypeStruct((MAX_SEQ, VOCAB_PAD, D_MODEL), jnp.float32),
)

SB = 16


def _tc_body(out_alias_ref, tok_ref, comb_ref, posoh_ref, out_ref):
    del out_alias_ref
    t = tok_ref[...]
    oh = (t == lax.broadcasted_iota(jnp.int32, (1, VOCAB_PAD), 1)).astype(
        jnp.bfloat16
    )
    ohc = jnp.concatenate([oh, posoh_ref[...]], axis=1)
    acc = jnp.dot(ohc, comb_ref[...], preferred_element_type=jnp.float32)
    out_ref[...] = jnp.reshape(acc, (SB, MAX_SEQ, D_MODEL))


_tc_fill = pl.pallas_call(
    _tc_body,
    grid=(B_TC // SB,),
    in_specs=[
        pl.BlockSpec(memory_space=pltpu.MemorySpace.HBM),
        pl.BlockSpec((SB * MAX_SEQ, 1), lambda b: (b, 0)),
        pl.BlockSpec((VOCAB_PAD + MAX_SEQ, D_MODEL), lambda b: (0, 0)),
        pl.BlockSpec((SB * MAX_SEQ, MAX_SEQ), lambda b: (0, 0)),
    ],
    out_specs=pl.BlockSpec(
        (SB, MAX_SEQ, D_MODEL), lambda b: (B_SC // SB + b, 0, 0)
    ),
    out_shape=jax.ShapeDtypeStruct((BATCH, MAX_SEQ, D_MODEL), jnp.float32),
    input_output_aliases={0: 0},
)

_mesh = plsc.VectorSubcoreMesh(core_axis_name="c", subcore_axis_name="s")


@functools.partial(
    pl.kernel,
    out_type=jax.ShapeDtypeStruct((N_ROWS, D_MODEL), jnp.float32),
    mesh=_mesh,
    scratch_types=[
        pltpu.VMEM((ROWS_PER_W,), jnp.int32),
        [pltpu.VMEM((CHUNK, D_MODEL), jnp.float32) for _ in range(NBUF)],
        [pltpu.SemaphoreType.DMA for _ in range(NBUF)],
        [pltpu.SemaphoreType.DMA for _ in range(NBUF)],
    ],
)
def _gather_kernel(tok_hbm, fused_hbm, out_hbm, idx_v, bufs, gsems, ssems):
    wid = lax.axis_index("s") * _NUM_CORES + lax.axis_index("c")
    base = wid * ROWS_PER_W
    pltpu.sync_copy(tok_hbm.at[pl.ds(base, ROWS_PER_W)], idx_v)

    lanes = lax.iota(jnp.int32, _LANES)

    def idx_body(j, carry):
        o = j * _LANES
        tok = idx_v[pl.ds(o, _LANES)]
        pos = jnp.remainder(o + lanes, MAX_SEQ)
        idx_v[pl.ds(o, _LANES)] = pos * VOCAB_PAD + tok
        return carry

    lax.fori_loop(0, ROWS_PER_W // _LANES, idx_body, 0)

    def fire_gather(c, b):
        pltpu.async_copy(
            fused_hbm.at[idx_v.at[pl.ds(c * CHUNK, CHUNK)]], bufs[b], gsems[b]
        )

    def wait_gather(b):
        pltpu.make_async_copy(
            out_hbm.at[pl.ds(base, CHUNK)], bufs[b], gsems[b]
        ).wait()

    def fire_scatter(c, b):
        pltpu.async_copy(
            bufs[b], out_hbm.at[pl.ds(base + c * CHUNK, CHUNK)], ssems[b]
        )

    def wait_scatter(b):
        pltpu.make_async_copy(
            bufs[b], out_hbm.at[pl.ds(base, CHUNK)], ssems[b]
        ).wait()

    fire_gather(0, 0)
    fire_gather(1, 1)
    for c in (0, 1, 2):
        wait_gather(c)
        fire_scatter(c, c)
        fire_gather(c + 2, c + 2)

    def chunk_body(g, carry):
        for k in range(NBUF):
            c = 3 + g * NBUF + k
            b = (3 + k) % NBUF
            b2 = (5 + k) % NBUF
            wait_gather(b)
            fire_scatter(c, b)
            wait_scatter(b2)
            fire_gather(c + 2, b2)
        return carry

    lax.fori_loop(0, (N_CHUNKS - 5) // NBUF, chunk_body, 0)

    for c in (N_CHUNKS - 2, N_CHUNKS - 1):
        b = c % NBUF
        wait_gather(b)
        fire_scatter(c, b)
    for b in range(NBUF):
        wait_scatter(b)


def kernel(token_ids, embedding_table):
    tok_flat = token_ids.reshape(-1).astype(jnp.int32)
    table_pad = jnp.pad(embedding_table, ((0, VOCAB_PAD - VOCAB), (0, 0)))
    pe = _positional_encoding()
    fused = _build_fused(table_pad, pe).reshape(MAX_SEQ * VOCAB_PAD, D_MODEL)
    sc_out = _gather_kernel(tok_flat, fused).reshape(BATCH, MAX_SEQ, D_MODEL)
    scaled = table_pad * math.sqrt(float(D_MODEL))
    comb = jnp.concatenate(
        [scaled.astype(jnp.bfloat16), pe.astype(jnp.bfloat16)], axis=0
    )
    rows = jnp.arange(SB * MAX_SEQ, dtype=jnp.int32)
    posoh = (
        (rows % MAX_SEQ)[:, None] == jnp.arange(MAX_SEQ, dtype=jnp.int32)[None]
    ).astype(jnp.bfloat16)
    tok_tc = token_ids[B_SC:].reshape(B_TC * MAX_SEQ, 1).astype(jnp.int32)
    return _tc_fill(sc_out, tok_tc, comb, posoh)


# speedup vs baseline: 1.9139x; 1.0177x over previous
import functools
import math

import jax
import jax.numpy as jnp
from jax import lax
from jax.experimental import pallas as pl
from jax.experimental.pallas import tpu as pltpu
from jax.experimental.pallas import tpu_sc as plsc

D_MODEL = 512
MAX_SEQ = 200
VOCAB = 76
VOCAB_PAD = 80
BATCH = 1024

_info = plsc.get_sparse_core_info()
_NUM_CORES = _info.num_cores
_NUM_SUBCORES = _info.num_subcores
_NUM_WORKERS = _NUM_CORES * _NUM_SUBCORES
_LANES = _info.num_lanes

N_ROWS = BATCH * MAX_SEQ
B_SC = 512
B_TC = BATCH - B_SC
ROWS_PER_W = B_SC * MAX_SEQ // _NUM_WORKERS
CHUNK = 40
N_CHUNKS = ROWS_PER_W // CHUNK
NBUF = 5


def _positional_encoding():
    even_i = jnp.arange(0, D_MODEL, 2, dtype=jnp.float32)
    denominator = jnp.power(10000.0, even_i / D_MODEL)
    position = jnp.arange(0, MAX_SEQ, 1, dtype=jnp.float32).reshape(MAX_SEQ, 1)
    even_pe = jnp.sin(position / denominator)
    odd_pe = jnp.cos(position / denominator)
    return jnp.stack([even_pe, odd_pe], axis=2).reshape(MAX_SEQ, D_MODEL)


def _fuse_body(table_ref, pe_ref, out_ref):
    out_ref[...] = (
        table_ref[...] * math.sqrt(float(D_MODEL)) + pe_ref[...][:, None, :]
    )


_build_fused = pl.pallas_call(
    _fuse_body,
    out_shape=jax.ShapeDtypeStruct((MAX_SEQ, VOCAB_PAD, D_MODEL), jnp.float32),
)

SB = 32


def _tc_body(out_alias_ref, tok_ref, comb_ref, posoh_ref, out_ref):
    del out_alias_ref
    t = tok_ref[...]
    oh = (t == lax.broadcasted_iota(jnp.int32, (1, VOCAB_PAD), 1)).astype(
        jnp.bfloat16
    )
    ohc = jnp.concatenate([oh, posoh_ref[...]], axis=1)
    acc = jnp.dot(ohc, comb_ref[...], preferred_element_type=jnp.float32)
    out_ref[...] = jnp.reshape(acc, (SB, MAX_SEQ, D_MODEL))


_tc_fill = pl.pallas_call(
    _tc_body,
    grid=(B_TC // SB,),
    in_specs=[
        pl.BlockSpec(memory_space=pltpu.MemorySpace.HBM),
        pl.BlockSpec((SB * MAX_SEQ, 1), lambda b: (b, 0)),
        pl.BlockSpec((VOCAB_PAD + MAX_SEQ, D_MODEL), lambda b: (0, 0)),
        pl.BlockSpec((SB * MAX_SEQ, MAX_SEQ), lambda b: (0, 0)),
    ],
    out_specs=pl.BlockSpec(
        (SB, MAX_SEQ, D_MODEL), lambda b: (B_SC // SB + b, 0, 0)
    ),
    out_shape=jax.ShapeDtypeStruct((BATCH, MAX_SEQ, D_MODEL), jnp.float32),
    input_output_aliases={0: 0},
)

_mesh = plsc.VectorSubcoreMesh(core_axis_name="c", subcore_axis_name="s")


@functools.partial(
    pl.kernel,
    out_type=jax.ShapeDtypeStruct((N_ROWS, D_MODEL), jnp.float32),
    mesh=_mesh,
    scratch_types=[
        pltpu.VMEM((ROWS_PER_W,), jnp.int32),
        [pltpu.VMEM((CHUNK, D_MODEL), jnp.float32) for _ in range(NBUF)],
        [pltpu.SemaphoreType.DMA for _ in range(NBUF)],
        [pltpu.SemaphoreType.DMA for _ in range(NBUF)],
    ],
)
def _gather_kernel(tok_hbm, fused_hbm, out_hbm, idx_v, bufs, gsems, ssems):
    wid = lax.axis_index("s") * _NUM_CORES + lax.axis_index("c")
    base = wid * ROWS_PER_W
    pltpu.sync_copy(tok_hbm.at[pl.ds(base, ROWS_PER_W)], idx_v)

    lanes = lax.iota(jnp.int32, _LANES)

    def idx_body(j, carry):
        o = j * _LANES
        tok = idx_v[pl.ds(o, _LANES)]
        pos = jnp.remainder(o + lanes, MAX_SEQ)
        idx_v[pl.ds(o, _LANES)] = pos * VOCAB_PAD + tok
        return carry

    lax.fori_loop(0, ROWS_PER_W // _LANES, idx_body, 0)

    def fire_gather(c, b):
        pltpu.async_copy(
            fused_hbm.at[idx_v.at[pl.ds(c * CHUNK, CHUNK)]], bufs[b], gsems[b]
        )

    def wait_gather(b):
        pltpu.make_async_copy(
            out_hbm.at[pl.ds(base, CHUNK)], bufs[b], gsems[b]
        ).wait()

    def fire_scatter(c, b):
        pltpu.async_copy(
            bufs[b], out_hbm.at[pl.ds(base + c * CHUNK, CHUNK)], ssems[b]
        )

    def wait_scatter(b):
        pltpu.make_async_copy(
            bufs[b], out_hbm.at[pl.ds(base, CHUNK)], ssems[b]
        ).wait()

    fire_gather(0, 0)
    fire_gather(1, 1)
    for c in (0, 1, 2):
        wait_gather(c)
        fire_scatter(c, c)
        fire_gather(c + 2, c + 2)

    def chunk_body(g, carry):
        for k in range(NBUF):
            c = 3 + g * NBUF + k
            b = (3 + k) % NBUF
            b2 = (5 + k) % NBUF
            wait_gather(b)
            fire_scatter(c, b)
            wait_scatter(b2)
            fire_gather(c + 2, b2)
        return carry

    lax.fori_loop(0, (N_CHUNKS - 5) // NBUF, chunk_body, 0)

    for c in (N_CHUNKS - 2, N_CHUNKS - 1):
        b = c % NBUF
        wait_gather(b)
        fire_scatter(c, b)
    for b in range(NBUF):
        wait_scatter(b)


def kernel(token_ids, embedding_table):
    tok_flat = token_ids.reshape(-1).astype(jnp.int32)
    table_pad = jnp.pad(embedding_table, ((0, VOCAB_PAD - VOCAB), (0, 0)))
    pe = _positional_encoding()
    fused = _build_fused(table_pad, pe).reshape(MAX_SEQ * VOCAB_PAD, D_MODEL)
    sc_out = _gather_kernel(tok_flat, fused).reshape(BATCH, MAX_SEQ, D_MODEL)
    scaled = table_pad * math.sqrt(float(D_MODEL))
    comb = jnp.concatenate(
        [scaled.astype(jnp.bfloat16), pe.astype(jnp.bfloat16)], axis=0
    )
    rows = jnp.arange(SB * MAX_SEQ, dtype=jnp.int32)
    posoh = (
        (rows % MAX_SEQ)[:, None] == jnp.arange(MAX_SEQ, dtype=jnp.int32)[None]
    ).astype(jnp.bfloat16)
    tok_tc = token_ids[B_SC:].reshape(B_TC * MAX_SEQ, 1).astype(jnp.int32)
    return _tc_fill(sc_out, tok_tc, comb, posoh)


# speedup vs baseline: 1.9636x; 1.0260x over previous
import functools
import math

import jax
import jax.numpy as jnp
from jax import lax
from jax.experimental import pallas as pl
from jax.experimental.pallas import tpu as pltpu
from jax.experimental.pallas import tpu_sc as plsc

D_MODEL = 512
MAX_SEQ = 200
VOCAB = 76
VOCAB_PAD = 80
BATCH = 1024

_info = plsc.get_sparse_core_info()
_NUM_CORES = _info.num_cores
_NUM_SUBCORES = _info.num_subcores
_NUM_WORKERS = _NUM_CORES * _NUM_SUBCORES
_LANES = _info.num_lanes

N_ROWS = BATCH * MAX_SEQ
B_SC = 448
B_TC = BATCH - B_SC
ROWS_PER_W = B_SC * MAX_SEQ // _NUM_WORKERS
CHUNK = 40
N_CHUNKS = ROWS_PER_W // CHUNK
NBUF = 5


def _positional_encoding():
    even_i = jnp.arange(0, D_MODEL, 2, dtype=jnp.float32)
    denominator = jnp.power(10000.0, even_i / D_MODEL)
    position = jnp.arange(0, MAX_SEQ, 1, dtype=jnp.float32).reshape(MAX_SEQ, 1)
    even_pe = jnp.sin(position / denominator)
    odd_pe = jnp.cos(position / denominator)
    return jnp.stack([even_pe, odd_pe], axis=2).reshape(MAX_SEQ, D_MODEL)


def _fuse_body(table_ref, pe_ref, out_ref):
    out_ref[...] = (
        table_ref[...] * math.sqrt(float(D_MODEL)) + pe_ref[...][:, None, :]
    )


_build_fused = pl.pallas_call(
    _fuse_body,
    out_shape=jax.ShapeDtypeStruct((MAX_SEQ, VOCAB_PAD, D_MODEL), jnp.float32),
)

SB = 32


def _tc_body(out_alias_ref, tok_ref, comb_ref, posoh_ref, out_ref):
    del out_alias_ref
    t = tok_ref[...]
    oh = (t == lax.broadcasted_iota(jnp.int32, (1, VOCAB_PAD), 1)).astype(
        jnp.bfloat16
    )
    ohc = jnp.concatenate([oh, posoh_ref[...]], axis=1)
    acc = jnp.dot(ohc, comb_ref[...], preferred_element_type=jnp.float32)
    out_ref[...] = jnp.reshape(acc, (SB, MAX_SEQ, D_MODEL))


_tc_fill = pl.pallas_call(
    _tc_body,
    grid=(B_TC // SB,),
    in_specs=[
        pl.BlockSpec(memory_space=pltpu.MemorySpace.HBM),
        pl.BlockSpec((SB * MAX_SEQ, 1), lambda b: (b, 0)),
        pl.BlockSpec((VOCAB_PAD + MAX_SEQ, D_MODEL), lambda b: (0, 0)),
        pl.BlockSpec((SB * MAX_SEQ, MAX_SEQ), lambda b: (0, 0)),
    ],
    out_specs=pl.BlockSpec(
        (SB, MAX_SEQ, D_MODEL), lambda b: (B_SC // SB + b, 0, 0)
    ),
    out_shape=jax.ShapeDtypeStruct((BATCH, MAX_SEQ, D_MODEL), jnp.float32),
    input_output_aliases={0: 0},
)

_mesh = plsc.VectorSubcoreMesh(core_axis_name="c", subcore_axis_name="s")


@functools.partial(
    pl.kernel,
    out_type=jax.ShapeDtypeStruct((N_ROWS, D_MODEL), jnp.float32),
    mesh=_mesh,
    scratch_types=[
        pltpu.VMEM((ROWS_PER_W,), jnp.int32),
        [pltpu.VMEM((CHUNK, D_MODEL), jnp.float32) for _ in range(NBUF)],
        [pltpu.SemaphoreType.DMA for _ in range(NBUF)],
        [pltpu.SemaphoreType.DMA for _ in range(NBUF)],
    ],
)
def _gather_kernel(tok_hbm, fused_hbm, out_hbm, idx_v, bufs, gsems, ssems):
    wid = lax.axis_index("s") * _NUM_CORES + lax.axis_index("c")
    base = wid * ROWS_PER_W
    pltpu.sync_copy(tok_hbm.at[pl.ds(base, ROWS_PER_W)], idx_v)

    lanes = lax.iota(jnp.int32, _LANES)

    def idx_body(j, carry):
        o = j * _LANES
        tok = idx_v[pl.ds(o, _LANES)]
        pos = jnp.remainder(o + lanes, MAX_SEQ)
        idx_v[pl.ds(o, _LANES)] = pos * VOCAB_PAD + tok
        return carry

    lax.fori_loop(0, ROWS_PER_W // _LANES, idx_body, 0)

    def fire_gather(c, b):
        pltpu.async_copy(
            fused_hbm.at[idx_v.at[pl.ds(c * CHUNK, CHUNK)]], bufs[b], gsems[b]
        )

    def wait_gather(b):
        pltpu.make_async_copy(
            out_hbm.at[pl.ds(base, CHUNK)], bufs[b], gsems[b]
        ).wait()

    def fire_scatter(c, b):
        pltpu.async_copy(
            bufs[b], out_hbm.at[pl.ds(base + c * CHUNK, CHUNK)], ssems[b]
        )

    def wait_scatter(b):
        pltpu.make_async_copy(
            bufs[b], out_hbm.at[pl.ds(base, CHUNK)], ssems[b]
        ).wait()

    fire_gather(0, 0)
    fire_gather(1, 1)
    for c in (0, 1, 2):
        wait_gather(c)
        fire_scatter(c, c)
        fire_gather(c + 2, c + 2)

    def chunk_body(g, carry):
        for k in range(NBUF):
            c = 3 + g * NBUF + k
            b = (3 + k) % NBUF
            b2 = (5 + k) % NBUF
            wait_gather(b)
            fire_scatter(c, b)
            wait_scatter(b2)
            fire_gather(c + 2, b2)
        return carry

    lax.fori_loop(0, (N_CHUNKS - 5) // NBUF, chunk_body, 0)

    for c in (N_CHUNKS - 2, N_CHUNKS - 1):
        b = c % NBUF
        wait_gather(b)
        fire_scatter(c, b)
    for b in range(NBUF):
        wait_scatter(b)


def kernel(token_ids, embedding_table):
    tok_flat = token_ids.reshape(-1).astype(jnp.int32)
    table_pad = jnp.pad(embedding_table, ((0, VOCAB_PAD - VOCAB), (0, 0)))
    pe = _positional_encoding()
    fused = _build_fused(table_pad, pe).reshape(MAX_SEQ * VOCAB_PAD, D_MODEL)
    sc_out = _gather_kernel(tok_flat, fused).reshape(BATCH, MAX_SEQ, D_MODEL)
    scaled = table_pad * math.sqrt(float(D_MODEL))
    comb = jnp.concatenate(
        [scaled.astype(jnp.bfloat16), pe.astype(jnp.bfloat16)], axis=0
    )
    rows = jnp.arange(SB * MAX_SEQ, dtype=jnp.int32)
    posoh = (
        (rows % MAX_SEQ)[:, None] == jnp.arange(MAX_SEQ, dtype=jnp.int32)[None]
    ).astype(jnp.bfloat16)
    tok_tc = token_ids[B_SC:].reshape(B_TC * MAX_SEQ, 1).astype(jnp.int32)
    return _tc_fill(sc_out, tok_tc, comb, posoh)
